# baseline (speedup 1.0000x reference)
import functools

import jax
import jax.numpy as jnp
from jax import lax
from jax.experimental import pallas as pl
from jax.experimental.pallas import tpu as pltpu
from jax.experimental.pallas import tpu_sc as plsc

N = 10000
E = 320000
D = 128
D_OUT = 64
NC = 2
NS = 16
NW = NC * NS
EPW = E // NW
K = 80
C = EPW // K
GC = 25
NG = C // GC
NPD = 10240
DEG_ROWS = NPD // NS
S_ROWS = NPD // NS

_MESH = dict(core_axis_name="c", subcore_axis_name="s",
             num_cores=NC, num_subcores=NS)


def _sc_kernel(src4, dst4, w4, h):
    @functools.partial(
        pl.kernel,
        out_type=(
            jax.ShapeDtypeStruct((NC, NPD, D), jnp.float32),
            jax.ShapeDtypeStruct((NC, NPD), jnp.float32),
        ),
        mesh=plsc.VectorSubcoreMesh(**_MESH),
        compiler_params=pltpu.CompilerParams(needs_layout_passes=False),
        scratch_types=[
            pltpu.VMEM((GC, K), jnp.int32),
            pltpu.VMEM((GC, K), jnp.int32),
            pltpu.VMEM((GC, K), jnp.float32),
            pltpu.VMEM((K, D), jnp.float32),
            pltpu.VMEM((K, D), jnp.float32),
            pltpu.VMEM((NPD,), jnp.float32),
            pltpu.VMEM((DEG_ROWS,), jnp.float32),
            pltpu.VMEM_SHARED((NPD,), jnp.float32),
            pltpu.VMEM_SHARED((NPD, D), jnp.float32),
            pltpu.SemaphoreType.DMA,
            pltpu.SemaphoreType.DMA,
            pltpu.SemaphoreType.DMA,
            pltpu.SemaphoreType.DMA,
        ],
    )
    def k(src4_h, dst4_h, w4_h, h_hbm, s_out, dis_out,
          sg_v, dg_v, wg_v, rows_a, rows_b, dis_v, nbuf,
          deg_sh, s_sh, semg_a, semg_b, sems_a, sems_b):
        c = lax.axis_index("c")
        s = lax.axis_index("s")
        wid = s * NC + c
        base = s * DEG_ROWS

        z16 = jnp.zeros((16,), jnp.float32)

        def zero_nbuf(i, carry):
            nbuf[pl.ds(i * 16, 16)] = z16
            return carry
        lax.fori_loop(0, DEG_ROWS // 16, zero_nbuf, 0)
        pltpu.sync_copy(nbuf, deg_sh.at[pl.ds(base, DEG_ROWS)])

        def zero_rows(i, carry):
            for t in range(D // 16):
                rows_a[i, pl.ds(t * 16, 16)] = z16
            return carry
        lax.fori_loop(0, K, zero_rows, 0)
        for kk in range(S_ROWS // K):
            pltpu.sync_copy(rows_a, s_sh.at[pl.ds(base + kk * K, K)])
        plsc.subcore_barrier()

        for t in range(2):
            slab = s * 2 + t
            for g in range(NG):
                pltpu.sync_copy(dst4_h.at[slab, g], dg_v)
                pltpu.sync_copy(w4_h.at[slab, g], wg_v)

                def deg_group(i, carry):
                    for u in range(5):
                        j = i * 5 + u
                        pltpu.async_copy(
                            wg_v.at[j], deg_sh.at[dg_v.at[j]], semg_a,
                            add=True)
                    for u in range(5):
                        j = i * 5 + u
                        pltpu.make_async_copy(
                            wg_v.at[j], deg_sh.at[dg_v.at[j]], semg_a).wait()
                    return carry
                lax.fori_loop(0, GC // 5, deg_group, 0)
        plsc.subcore_barrier()

        pltpu.sync_copy(deg_sh.at[pl.ds(base, DEG_ROWS)], nbuf)

        def newton(i, carry):
            d = nbuf[pl.ds(i * 16, 16)] + 1.0
            bits = lax.bitcast_convert_type(d, jnp.int32)
            bits = jnp.int32(0x5F3759DF) - lax.shift_right_logical(bits, 1)
            y = lax.bitcast_convert_type(bits, jnp.float32)
            for _ in range(3):
                y = y * (1.5 - 0.5 * d * y * y)
            nbuf[pl.ds(i * 16, 16)] = y
            return carry
        lax.fori_loop(0, DEG_ROWS // 16, newton, 0)
        pltpu.sync_copy(nbuf, dis_out.at[c, pl.ds(base, DEG_ROWS)])
        plsc.subcore_barrier()

        pltpu.sync_copy(dis_out.at[c], dis_v)

        def g_start(u, buf, p, sem):
            pltpu.async_copy(h_hbm.at[sg_v.at[u]], buf, sem)

        def g_wait(u, buf, p, sem):
            pltpu.make_async_copy(h_hbm.at[sg_v.at[u]], buf, sem).wait()

        def s_start(u, buf, sem):
            pltpu.async_copy(buf, s_sh.at[dg_v.at[u]], sem, add=True)

        def s_wait(u, buf, sem):
            pltpu.make_async_copy(buf, s_sh.at[dg_v.at[u]], sem).wait()

        def scale(u, buf, p):
            def cfold(kk, carry):
                s16 = sg_v[u, pl.ds(kk * 16, 16)]
                wg_v[u, pl.ds(kk * 16, 16)] = (
                    wg_v[u, pl.ds(kk * 16, 16)] * plsc.load_gather(
                        dis_v, [s16]))
                return carry
            lax.fori_loop(0, K // 16, cfold, 0)

            def row(i, carry):
                ii = jnp.full((16,), i, jnp.int32)
                uu = jnp.full((16,), u, jnp.int32)
                cb = plsc.load_gather(wg_v, [uu, ii])
                for t in range(D // 16):
                    buf[i, pl.ds(t * 16, 16)] = buf[i, pl.ds(t * 16, 16)] * cb
                return carry
            lax.fori_loop(0, K, row, 0)

        for g in range(NG):
            pltpu.sync_copy(src4_h.at[wid, g], sg_v)
            pltpu.sync_copy(dst4_h.at[wid, g], dg_v)
            pltpu.sync_copy(w4_h.at[wid, g], wg_v)
            g_start(0, rows_a, 0, semg_a)

            def pair(tt, carry):
                u0 = 2 * tt
                u1 = u0 + 1
                g_start(u1, rows_b, 1, semg_b)
                g_wait(u0, rows_a, 0, semg_a)
                scale(u0, rows_a, 0)
                s_start(u0, rows_a, sems_a)
                g_wait(u1, rows_b, 1, semg_b)
                scale(u1, rows_b, 1)
                s_start(u1, rows_b, sems_b)
                s_wait(u0, rows_a, sems_a)
                g_start(u0 + 2, rows_a, 0, semg_a)
                s_wait(u1, rows_b, sems_b)
                return carry
            lax.fori_loop(0, (GC - 1) // 2, pair, 0)

            ul = GC - 1
            g_wait(ul, rows_a, 0, semg_a)
            scale(ul, rows_a, 0)
            s_start(ul, rows_a, sems_a)
            s_wait(ul, rows_a, sems_a)
        plsc.subcore_barrier()

        pltpu.sync_copy(s_sh.at[pl.ds(base, S_ROWS)],
                        s_out.at[c, pl.ds(base, S_ROWS)])

    return k(src4, dst4, w4, h)


def _mm_h(x, W_conv):
    def body(x_ref, w_ref, h_ref):
        h_ref[...] = jnp.dot(x_ref[...], w_ref[...],
                             preferred_element_type=jnp.float32)

    return pl.pallas_call(
        body,
        grid=(N // 400,),
        in_specs=[
            pl.BlockSpec((400, D), lambda i: (i, 0)),
            pl.BlockSpec((D, D), lambda i: (0, 0)),
        ],
        out_specs=pl.BlockSpec((400, D), lambda i: (i, 0)),
        out_shape=jax.ShapeDtypeStruct((N, D), jnp.float32),
    )(x, W_conv)


def _epilogue(s0, s1, h, dis, b_conv, W_out, b_out):
    def body(s0_ref, s1_ref, h_ref, dis_ref, bc_ref, wo_ref, bo_ref, o_ref):
        dv = dis_ref[...]
        agg = dv * (s0_ref[...] + s1_ref[...]) + (dv * dv) * h_ref[...] \
            + bc_ref[...]
        emb = jnp.where(agg >= 0, agg, 0.01 * agg)
        o_ref[...] = jnp.dot(emb, wo_ref[...],
                             preferred_element_type=jnp.float32) + bo_ref[...]

    return pl.pallas_call(
        body,
        grid=(N // 400,),
        in_specs=[
            pl.BlockSpec((400, D), lambda i: (i, 0)),
            pl.BlockSpec((400, D), lambda i: (i, 0)),
            pl.BlockSpec((400, D), lambda i: (i, 0)),
            pl.BlockSpec((400, 1), lambda i: (i, 0)),
            pl.BlockSpec((1, D), lambda i: (0, 0)),
            pl.BlockSpec((D, D_OUT), lambda i: (0, 0)),
            pl.BlockSpec((1, D_OUT), lambda i: (0, 0)),
        ],
        out_specs=pl.BlockSpec((400, D_OUT), lambda i: (i, 0)),
        out_shape=jax.ShapeDtypeStruct((N, D_OUT), jnp.float32),
    )(s0, s1, h, dis, b_conv, W_out, b_out)


def kernel(x, edge_index, edge_weight, W_conv, b_conv, W_out, b_out):
    src = edge_index[0].astype(jnp.int32)
    dst = edge_index[1].astype(jnp.int32)
    w = edge_weight.astype(jnp.float32)

    src4 = src.reshape(NW, NG, GC, K)
    dst4 = dst.reshape(NW, NG, GC, K)
    w4 = w.reshape(NW, NG, GC, K)

    h = _mm_h(x, W_conv)
    s_part, dis = _sc_kernel(src4, dst4, w4, h)
    out = _epilogue(
        s_part[0, :N], s_part[1, :N], h, dis[0, :N].reshape(N, 1),
        b_conv.reshape(1, D), W_out, b_out.reshape(1, D_OUT))
    return out


# speedup vs baseline: 1.0875x; 1.0875x over previous
import functools

import jax
import jax.numpy as jnp
from jax import lax
from jax.experimental import pallas as pl
from jax.experimental.pallas import tpu as pltpu
from jax.experimental.pallas import tpu_sc as plsc

N = 10000
E = 320000
D = 128
D_OUT = 64
NC = 2
NS = 16
NW = NC * NS
EPW = E // NW
K = 80
C = EPW // K
GC = 25
NG = C // GC
NPD = 10240
DEG_ROWS = NPD // NS
S_ROWS = NPD // NS

_MESH = dict(core_axis_name="c", subcore_axis_name="s",
             num_cores=NC, num_subcores=NS)
_PARAMS = None


def _sc_params():
    return pltpu.CompilerParams(needs_layout_passes=False)


def _deg_kernel(dst3, w3):
    @functools.partial(
        pl.kernel,
        out_type=jax.ShapeDtypeStruct((NC, NPD), jnp.float32),
        mesh=plsc.VectorSubcoreMesh(**_MESH),
        compiler_params=_sc_params(),
        scratch_types=[
            pltpu.VMEM((NG, GC, K), jnp.int32),
            pltpu.VMEM((NG, GC, K), jnp.float32),
            pltpu.VMEM((DEG_ROWS,), jnp.float32),
            pltpu.VMEM_SHARED((NPD,), jnp.float32),
            pltpu.SemaphoreType.DMA,
        ],
    )
    def k(dst3_h, w3_h, deg_out, dst_v, w_v, nbuf, deg_sh, sem):
        c = lax.axis_index("c")
        s = lax.axis_index("s")
        wid = s * NC + c
        base = s * DEG_ROWS

        z16 = jnp.zeros((16,), jnp.float32)

        def zero_nbuf(i, carry):
            nbuf[pl.ds(i * 16, 16)] = z16
            return carry
        lax.fori_loop(0, DEG_ROWS // 16, zero_nbuf, 0)
        pltpu.sync_copy(nbuf, deg_sh.at[pl.ds(base, DEG_ROWS)])

        pltpu.sync_copy(dst3_h.at[wid], dst_v)
        pltpu.sync_copy(w3_h.at[wid], w_v)
        plsc.subcore_barrier()

        def deg_group(i, carry):
            g = i // 5
            t = i % 5
            for u in range(5):
                j = t * 5 + u
                pltpu.async_copy(
                    w_v.at[g, j], deg_sh.at[dst_v.at[g, j]], sem, add=True)
            for u in range(5):
                j = t * 5 + u
                pltpu.make_async_copy(
                    w_v.at[g, j], deg_sh.at[dst_v.at[g, j]], sem).wait()
            return carry
        lax.fori_loop(0, C // 5, deg_group, 0)
        plsc.subcore_barrier()

        pltpu.sync_copy(deg_sh.at[pl.ds(base, DEG_ROWS)],
                        deg_out.at[c, pl.ds(base, DEG_ROWS)])

    return k(dst3, w3)


def _agg_kernel(src3, dst3, w3, hp):
    @functools.partial(
        pl.kernel,
        out_type=jax.ShapeDtypeStruct((NC, NPD, D), jnp.float32),
        mesh=plsc.VectorSubcoreMesh(**_MESH),
        compiler_params=_sc_params(),
        scratch_types=[
            pltpu.VMEM((GC, K), jnp.int32),
            pltpu.VMEM((GC, K), jnp.int32),
            pltpu.VMEM((GC, K), jnp.float32),
            pltpu.VMEM((K, D), jnp.float32),
            pltpu.VMEM((K, D), jnp.float32),
            pltpu.VMEM_SHARED((NPD, D), jnp.float32),
            pltpu.SemaphoreType.DMA,
            pltpu.SemaphoreType.DMA,
            pltpu.SemaphoreType.DMA,
            pltpu.SemaphoreType.DMA,
        ],
    )
    def k(src3_h, dst3_h, w3_h, hp_hbm, s_out,
          sg_v, dg_v, wg_v, rows_a, rows_b, s_sh,
          semg_a, semg_b, sems_a, sems_b):
        c = lax.axis_index("c")
        s = lax.axis_index("s")
        wid = s * NC + c
        base = s * S_ROWS

        z16 = jnp.zeros((16,), jnp.float32)

        def zero_rows(i, carry):
            for t in range(D // 16):
                rows_a[i, pl.ds(t * 16, 16)] = z16
            return carry
        lax.fori_loop(0, K, zero_rows, 0)
        for kk in range(S_ROWS // K):
            pltpu.sync_copy(rows_a, s_sh.at[pl.ds(base + kk * K, K)])
        plsc.subcore_barrier()

        def g_start(u, buf, sem):
            pltpu.async_copy(hp_hbm.at[sg_v.at[u]], buf, sem)

        def g_wait(u, buf, sem):
            pltpu.make_async_copy(hp_hbm.at[sg_v.at[u]], buf, sem).wait()

        def s_start(u, buf, sem):
            pltpu.async_copy(buf, s_sh.at[dg_v.at[u]], sem, add=True)

        def s_wait(u, buf, sem):
            pltpu.make_async_copy(buf, s_sh.at[dg_v.at[u]], sem).wait()

        def scale(u, buf):
            def row(i, carry):
                ii = jnp.full((16,), i, jnp.int32)
                uu = jnp.full((16,), u, jnp.int32)
                cb = plsc.load_gather(wg_v, [uu, ii])
                for t in range(D // 16):
                    buf[i, pl.ds(t * 16, 16)] = buf[i, pl.ds(t * 16, 16)] * cb
                return carry
            lax.fori_loop(0, K, row, 0)

        for g in range(NG):
            pltpu.sync_copy(src3_h.at[wid, g], sg_v)
            pltpu.sync_copy(dst3_h.at[wid, g], dg_v)
            pltpu.sync_copy(w3_h.at[wid, g], wg_v)
            g_start(0, rows_a, semg_a)

            def pair(tt, carry):
                u0 = 2 * tt
                u1 = u0 + 1
                g_start(u1, rows_b, semg_b)
                g_wait(u0, rows_a, semg_a)
                scale(u0, rows_a)
                s_start(u0, rows_a, sems_a)
                g_wait(u1, rows_b, semg_b)
                scale(u1, rows_b)
                s_start(u1, rows_b, sems_b)
                s_wait(u0, rows_a, sems_a)
                g_start(u0 + 2, rows_a, semg_a)
                s_wait(u1, rows_b, sems_b)
                return carry
            lax.fori_loop(0, (GC - 1) // 2, pair, 0)

            ul = GC - 1
            g_wait(ul, rows_a, semg_a)
            scale(ul, rows_a)
            s_start(ul, rows_a, sems_a)
            s_wait(ul, rows_a, sems_a)
        plsc.subcore_barrier()

        pltpu.sync_copy(s_sh.at[pl.ds(base, S_ROWS)],
                        s_out.at[c, pl.ds(base, S_ROWS)])

    return k(src3, dst3, w3, hp)


def _mm_h(x, W_conv, deg0, deg1):
    def body(x_ref, w_ref, d0_ref, d1_ref, hp_ref, dis_ref):
        deg = d0_ref[...] + d1_ref[...] + 1.0
        dis = lax.rsqrt(deg)
        hp_ref[...] = jnp.dot(x_ref[...], w_ref[...],
                              preferred_element_type=jnp.float32) * dis
        dis_ref[...] = dis

    return pl.pallas_call(
        body,
        grid=(N // 400,),
        in_specs=[
            pl.BlockSpec((400, D), lambda i: (i, 0)),
            pl.BlockSpec((D, D), lambda i: (0, 0)),
            pl.BlockSpec((400, 1), lambda i: (i, 0)),
            pl.BlockSpec((400, 1), lambda i: (i, 0)),
        ],
        out_specs=[
            pl.BlockSpec((400, D), lambda i: (i, 0)),
            pl.BlockSpec((400, 1), lambda i: (i, 0)),
        ],
        out_shape=[
            jax.ShapeDtypeStruct((N, D), jnp.float32),
            jax.ShapeDtypeStruct((N, 1), jnp.float32),
        ],
    )(x, W_conv, deg0, deg1)


def _epilogue(s_part, hp, dis, b_conv, W_out, b_out):
    def body(s0_ref, s1_ref, hp_ref, dis_ref, bc_ref, wo_ref, bo_ref, o_ref):
        agg = dis_ref[...] * (s0_ref[0] + s1_ref[0] + hp_ref[...]) \
            + bc_ref[...]
        emb = jnp.where(agg >= 0, agg, 0.01 * agg)
        o_ref[...] = jnp.dot(emb, wo_ref[...],
                             preferred_element_type=jnp.float32) + bo_ref[...]

    return pl.pallas_call(
        body,
        grid=(N // 400,),
        in_specs=[
            pl.BlockSpec((1, 400, D), lambda i: (0, i, 0)),
            pl.BlockSpec((1, 400, D), lambda i: (1, i, 0)),
            pl.BlockSpec((400, D), lambda i: (i, 0)),
            pl.BlockSpec((400, 1), lambda i: (i, 0)),
            pl.BlockSpec((1, D), lambda i: (0, 0)),
            pl.BlockSpec((D, D_OUT), lambda i: (0, 0)),
            pl.BlockSpec((1, D_OUT), lambda i: (0, 0)),
        ],
        out_specs=pl.BlockSpec((400, D_OUT), lambda i: (i, 0)),
        out_shape=jax.ShapeDtypeStruct((N, D_OUT), jnp.float32),
    )(s_part, s_part, hp, dis, b_conv, W_out, b_out)


def kernel(x, edge_index, edge_weight, W_conv, b_conv, W_out, b_out):
    src = edge_index[0].astype(jnp.int32)
    dst = edge_index[1].astype(jnp.int32)
    w = edge_weight.astype(jnp.float32)

    src4 = src.reshape(NW, NG, GC, K)
    dst4 = dst.reshape(NW, NG, GC, K)
    w4 = w.reshape(NW, NG, GC, K)

    deg_p = _deg_kernel(dst4, w4)
    deg0 = deg_p[0, :N].reshape(N, 1)
    deg1 = deg_p[1, :N].reshape(N, 1)
    hp, dis = _mm_h(x, W_conv, deg0, deg1)
    s_part = _agg_kernel(src4, dst4, w4, hp)
    out = _epilogue(
        s_part, hp, dis,
        b_conv.reshape(1, D), W_out, b_out.reshape(1, D_OUT))
    return out


# speedup vs baseline: 1.2079x; 1.1108x over previous
import functools

import jax
import jax.numpy as jnp
from jax import lax
from jax.experimental import pallas as pl
from jax.experimental.pallas import tpu as pltpu
from jax.experimental.pallas import tpu_sc as plsc

N = 10000
E = 320000
D = 128
D_OUT = 64
NC = 2
NS = 16
NW = NC * NS
EPW = E // NW
K = 80
C = EPW // K
GC = 25
NG = C // GC
NPD = 10240
DEG_ROWS = NPD // NS
S_ROWS = NPD // NS

_MESH = dict(core_axis_name="c", subcore_axis_name="s",
             num_cores=NC, num_subcores=NS)
_PARAMS = None


def _sc_params():
    return pltpu.CompilerParams(needs_layout_passes=False)


def _deg_kernel(ei4, w4):
    @functools.partial(
        pl.kernel,
        out_type=jax.ShapeDtypeStruct((NC, NPD), jnp.float32),
        mesh=plsc.VectorSubcoreMesh(**_MESH),
        compiler_params=_sc_params(),
        scratch_types=[
            pltpu.VMEM((NG, GC, K), jnp.int32),
            pltpu.VMEM((NG, GC, K), jnp.float32),
            pltpu.VMEM((DEG_ROWS,), jnp.float32),
            pltpu.VMEM_SHARED((NPD,), jnp.float32),
            pltpu.SemaphoreType.DMA,
        ],
    )
    def k(ei4_h, w4_h, deg_out, dst_v, w_v, nbuf, deg_sh, sem):
        c = lax.axis_index("c")
        s = lax.axis_index("s")
        wid = s * NC + c
        base = s * DEG_ROWS

        z16 = jnp.zeros((16,), jnp.float32)

        def zero_nbuf(i, carry):
            nbuf[pl.ds(i * 16, 16)] = z16
            return carry
        lax.fori_loop(0, DEG_ROWS // 16, zero_nbuf, 0)
        pltpu.sync_copy(nbuf, deg_sh.at[pl.ds(base, DEG_ROWS)])

        pltpu.sync_copy(ei4_h.at[1, wid], dst_v)
        pltpu.sync_copy(w4_h.at[wid], w_v)
        plsc.subcore_barrier()

        def deg_group(i, carry):
            g = i // 5
            t = i % 5
            for u in range(5):
                j = t * 5 + u
                pltpu.async_copy(
                    w_v.at[g, j], deg_sh.at[dst_v.at[g, j]], sem, add=True)
            for u in range(5):
                j = t * 5 + u
                pltpu.make_async_copy(
                    w_v.at[g, j], deg_sh.at[dst_v.at[g, j]], sem).wait()
            return carry
        lax.fori_loop(0, C // 5, deg_group, 0)
        plsc.subcore_barrier()

        pltpu.sync_copy(deg_sh.at[pl.ds(base, DEG_ROWS)],
                        deg_out.at[c, pl.ds(base, DEG_ROWS)])

    return k(ei4, w4)


def _agg_kernel(ei4, w4, hp):
    @functools.partial(
        pl.kernel,
        out_type=jax.ShapeDtypeStruct((NC, NPD, D), jnp.float32),
        mesh=plsc.VectorSubcoreMesh(**_MESH),
        compiler_params=_sc_params(),
        scratch_types=[
            pltpu.VMEM((GC, K), jnp.int32),
            pltpu.VMEM((GC, K), jnp.int32),
            pltpu.VMEM((GC, K), jnp.float32),
            pltpu.VMEM((K, D), jnp.float32),
            pltpu.VMEM((K, D), jnp.float32),
            pltpu.VMEM_SHARED((NPD, D), jnp.float32),
            pltpu.SemaphoreType.DMA,
            pltpu.SemaphoreType.DMA,
            pltpu.SemaphoreType.DMA,
            pltpu.SemaphoreType.DMA,
        ],
    )
    def k(ei4_h, w4_h, hp_hbm, s_out,
          sg_v, dg_v, wg_v, rows_a, rows_b, s_sh,
          semg_a, semg_b, sems_a, sems_b):
        c = lax.axis_index("c")
        s = lax.axis_index("s")
        wid = s * NC + c
        base = s * S_ROWS

        z16 = jnp.zeros((16,), jnp.float32)

        def zero_rows(i, carry):
            for t in range(D // 16):
                rows_a[i, pl.ds(t * 16, 16)] = z16
            return carry
        lax.fori_loop(0, K, zero_rows, 0)
        for kk in range(S_ROWS // K):
            pltpu.sync_copy(rows_a, s_sh.at[pl.ds(base + kk * K, K)])
        plsc.subcore_barrier()

        def g_start(u, buf, sem):
            pltpu.async_copy(hp_hbm.at[sg_v.at[u]], buf, sem)

        def g_wait(u, buf, sem):
            pltpu.make_async_copy(hp_hbm.at[sg_v.at[u]], buf, sem).wait()

        def s_start(u, buf, sem):
            pltpu.async_copy(buf, s_sh.at[dg_v.at[u]], sem, add=True)

        def s_wait(u, buf, sem):
            pltpu.make_async_copy(buf, s_sh.at[dg_v.at[u]], sem).wait()

        def scale(u, buf):
            def row(i, carry):
                ii = jnp.full((16,), i, jnp.int32)
                uu = jnp.full((16,), u, jnp.int32)
                cb = plsc.load_gather(wg_v, [uu, ii])
                for t in range(D // 16):
                    buf[i, pl.ds(t * 16, 16)] = buf[i, pl.ds(t * 16, 16)] * cb
                return carry
            lax.fori_loop(0, K, row, 0)

        for g in range(NG):
            pltpu.sync_copy(ei4_h.at[0, wid, g], sg_v)
            pltpu.sync_copy(ei4_h.at[1, wid, g], dg_v)
            pltpu.sync_copy(w4_h.at[wid, g], wg_v)
            g_start(0, rows_a, semg_a)

            def pair(tt, carry):
                u0 = 2 * tt
                u1 = u0 + 1
                g_start(u1, rows_b, semg_b)
                g_wait(u0, rows_a, semg_a)
                scale(u0, rows_a)
                s_start(u0, rows_a, sems_a)
                g_wait(u1, rows_b, semg_b)
                scale(u1, rows_b)
                s_start(u1, rows_b, sems_b)
                s_wait(u0, rows_a, sems_a)
                g_start(u0 + 2, rows_a, semg_a)
                s_wait(u1, rows_b, sems_b)
                return carry
            lax.fori_loop(0, (GC - 1) // 2, pair, 0)

            ul = GC - 1
            g_wait(ul, rows_a, semg_a)
            scale(ul, rows_a)
            s_start(ul, rows_a, sems_a)
            s_wait(ul, rows_a, sems_a)
        plsc.subcore_barrier()

        pltpu.sync_copy(s_sh.at[pl.ds(base, S_ROWS)],
                        s_out.at[c, pl.ds(base, S_ROWS)])

    return k(ei4, w4, hp)


def _mm_h(x, W_conv, deg0, deg1):
    def body(x_ref, w_ref, d0_ref, d1_ref, hp_ref, dis_ref):
        deg = d0_ref[...] + d1_ref[...] + 1.0
        dis = lax.rsqrt(deg)
        hp_ref[...] = jnp.dot(x_ref[...], w_ref[...],
                              preferred_element_type=jnp.float32) * dis
        dis_ref[...] = dis

    return pl.pallas_call(
        body,
        grid=(N // 2000,),
        in_specs=[
            pl.BlockSpec((2000, D), lambda i: (i, 0)),
            pl.BlockSpec((D, D), lambda i: (0, 0)),
            pl.BlockSpec((2000, 1), lambda i: (i, 0)),
            pl.BlockSpec((2000, 1), lambda i: (i, 0)),
        ],
        out_specs=[
            pl.BlockSpec((2000, D), lambda i: (i, 0)),
            pl.BlockSpec((2000, 1), lambda i: (i, 0)),
        ],
        out_shape=[
            jax.ShapeDtypeStruct((N, D), jnp.float32),
            jax.ShapeDtypeStruct((N, 1), jnp.float32),
        ],
    )(x, W_conv, deg0, deg1)


def _epilogue(s_part, hp, dis, b_conv, W_out, b_out):
    def body(s0_ref, s1_ref, hp_ref, dis_ref, bc_ref, wo_ref, bo_ref, o_ref):
        agg = dis_ref[...] * (s0_ref[0] + s1_ref[0] + hp_ref[...]) \
            + bc_ref[...]
        emb = jnp.where(agg >= 0, agg, 0.01 * agg)
        o_ref[...] = jnp.dot(emb, wo_ref[...],
                             preferred_element_type=jnp.float32) + bo_ref[...]

    return pl.pallas_call(
        body,
        grid=(N // 2000,),
        in_specs=[
            pl.BlockSpec((1, 2000, D), lambda i: (0, i, 0)),
            pl.BlockSpec((1, 2000, D), lambda i: (1, i, 0)),
            pl.BlockSpec((2000, D), lambda i: (i, 0)),
            pl.BlockSpec((2000, 1), lambda i: (i, 0)),
            pl.BlockSpec((1, D), lambda i: (0, 0)),
            pl.BlockSpec((D, D_OUT), lambda i: (0, 0)),
            pl.BlockSpec((1, D_OUT), lambda i: (0, 0)),
        ],
        out_specs=pl.BlockSpec((2000, D_OUT), lambda i: (i, 0)),
        out_shape=jax.ShapeDtypeStruct((N, D_OUT), jnp.float32),
    )(s_part, s_part, hp, dis, b_conv, W_out, b_out)


def kernel(x, edge_index, edge_weight, W_conv, b_conv, W_out, b_out):
    ei4 = edge_index.astype(jnp.int32).reshape(2, NW, NG, GC, K)
    w4 = edge_weight.astype(jnp.float32).reshape(NW, NG, GC, K)

    deg_p = _deg_kernel(ei4, w4)
    deg0 = deg_p[0].reshape(NPD, 1)
    deg1 = deg_p[1].reshape(NPD, 1)
    hp, dis = _mm_h(x, W_conv, deg0, deg1)
    s_part = _agg_kernel(ei4, w4, hp)
    out = _epilogue(
        s_part, hp, dis,
        b_conv.reshape(1, D), W_out, b_out.reshape(1, D_OUT))
    return out


# speedup vs baseline: 1.2474x; 1.0327x over previous
import functools

import jax
import jax.numpy as jnp
from jax import lax
from jax.experimental import pallas as pl
from jax.experimental.pallas import tpu as pltpu
from jax.experimental.pallas import tpu_sc as plsc

N = 10000
E = 320000
D = 128
D_OUT = 64
NC = 2
NS = 16
NW = NC * NS
EPW = E // NW
K = 80
C = EPW // K
GC = 25
NG = C // GC
NPD = 10240
DEG_ROWS = NPD // NS
S_ROWS = NPD // NS

_MESH = dict(core_axis_name="c", subcore_axis_name="s",
             num_cores=NC, num_subcores=NS)
_PARAMS = None


def _sc_params():
    return pltpu.CompilerParams(needs_layout_passes=False)


def _deg_kernel(ei4, w4):
    @functools.partial(
        pl.kernel,
        out_type=jax.ShapeDtypeStruct((NC, NPD), jnp.float32),
        mesh=plsc.VectorSubcoreMesh(**_MESH),
        compiler_params=_sc_params(),
        scratch_types=[
            pltpu.VMEM((NG, GC, K), jnp.int32),
            pltpu.VMEM((NG, GC, K), jnp.float32),
            pltpu.VMEM((DEG_ROWS,), jnp.float32),
            pltpu.VMEM_SHARED((NPD,), jnp.float32),
            pltpu.SemaphoreType.DMA,
        ],
    )
    def k(ei4_h, w4_h, deg_out, dst_v, w_v, nbuf, deg_sh, sem):
        c = lax.axis_index("c")
        s = lax.axis_index("s")
        wid = s * NC + c
        base = s * DEG_ROWS

        z16 = jnp.zeros((16,), jnp.float32)

        def zero_nbuf(i, carry):
            nbuf[pl.ds(i * 16, 16)] = z16
            return carry
        lax.fori_loop(0, DEG_ROWS // 16, zero_nbuf, 0)
        pltpu.sync_copy(nbuf, deg_sh.at[pl.ds(base, DEG_ROWS)])

        pltpu.sync_copy(ei4_h.at[1, wid], dst_v)
        pltpu.sync_copy(w4_h.at[wid], w_v)
        plsc.subcore_barrier()

        def deg_group(i, carry):
            g = i // 5
            t = i % 5
            for u in range(5):
                j = t * 5 + u
                pltpu.async_copy(
                    w_v.at[g, j], deg_sh.at[dst_v.at[g, j]], sem, add=True)
            for u in range(5):
                j = t * 5 + u
                pltpu.make_async_copy(
                    w_v.at[g, j], deg_sh.at[dst_v.at[g, j]], sem).wait()
            return carry
        lax.fori_loop(0, C // 5, deg_group, 0)
        plsc.subcore_barrier()

        pltpu.sync_copy(deg_sh.at[pl.ds(base, DEG_ROWS)],
                        deg_out.at[c, pl.ds(base, DEG_ROWS)])

    return k(ei4, w4)


def _agg_kernel(ei4, w4, hp):
    @functools.partial(
        pl.kernel,
        out_type=jax.ShapeDtypeStruct((NC, NPD, D), jnp.float32),
        mesh=plsc.VectorSubcoreMesh(**_MESH),
        compiler_params=_sc_params(),
        scratch_types=[
            pltpu.VMEM((2, GC, K), jnp.int32),
            pltpu.VMEM((2, GC, K), jnp.int32),
            pltpu.VMEM((2, GC, K), jnp.float32),
            pltpu.VMEM((K, D), jnp.float32),
            pltpu.VMEM((K, D), jnp.float32),
            pltpu.VMEM_SHARED((NPD, D), jnp.float32),
            pltpu.SemaphoreType.DMA,
            pltpu.SemaphoreType.DMA,
            pltpu.SemaphoreType.DMA,
            pltpu.SemaphoreType.DMA,
            pltpu.SemaphoreType.DMA,
        ],
    )
    def k(ei4_h, w4_h, hp_hbm, s_out,
          sg2, dg2, wg2, rows_a, rows_b, s_sh,
          semg_a, semg_b, sems_a, sems_b, semi):
        c = lax.axis_index("c")
        s = lax.axis_index("s")
        wid = s * NC + c
        base = s * S_ROWS

        z16 = jnp.zeros((16,), jnp.float32)

        def zero_rows(i, carry):
            for t in range(D // 16):
                rows_a[i, pl.ds(t * 16, 16)] = z16
            return carry
        lax.fori_loop(0, K, zero_rows, 0)
        for kk in range(S_ROWS // K):
            pltpu.sync_copy(rows_a, s_sh.at[pl.ds(base + kk * K, K)])
        plsc.subcore_barrier()

        def g_start(sg_v, u, buf, sem):
            pltpu.async_copy(hp_hbm.at[sg_v.at[u]], buf, sem)

        def g_wait(sg_v, u, buf, sem):
            pltpu.make_async_copy(hp_hbm.at[sg_v.at[u]], buf, sem).wait()

        def s_start(dg_v, u, buf, sem):
            pltpu.async_copy(buf, s_sh.at[dg_v.at[u]], sem, add=True)

        def s_wait(dg_v, u, buf, sem):
            pltpu.make_async_copy(buf, s_sh.at[dg_v.at[u]], sem).wait()

        def make_scale(wg_v):
            def scale(u, buf):
                def row(i, carry):
                    ii = jnp.full((16,), i, jnp.int32)
                    uu = jnp.full((16,), u, jnp.int32)
                    cb = plsc.load_gather(wg_v, [uu, ii])
                    for t in range(D // 16):
                        buf[i, pl.ds(t * 16, 16)] = \
                            buf[i, pl.ds(t * 16, 16)] * cb
                    return carry
                lax.fori_loop(0, K, row, 0)
            return scale

        def idx_prefetch(g, p, sem):
            pltpu.async_copy(ei4_h.at[0, wid, g], sg2.at[p], sem)
            pltpu.async_copy(ei4_h.at[1, wid, g], dg2.at[p], sem)
            pltpu.async_copy(w4_h.at[wid, g], wg2.at[p], sem)

        def idx_wait(g, p, sem):
            pltpu.make_async_copy(ei4_h.at[0, wid, g], sg2.at[p], sem).wait()
            pltpu.make_async_copy(ei4_h.at[1, wid, g], dg2.at[p], sem).wait()
            pltpu.make_async_copy(w4_h.at[wid, g], wg2.at[p], sem).wait()

        idx_prefetch(0, 0, semi)
        idx_wait(0, 0, semi)
        for g in range(NG):
            p = g % 2
            sg_v, dg_v, wg_v = sg2.at[p], dg2.at[p], wg2.at[p]
            scale = make_scale(wg_v)
            if g + 1 < NG:
                idx_prefetch(g + 1, 1 - p, semi)
            g_start(sg_v, 0, rows_a, semg_a)

            def pair(tt, carry):
                u0 = 2 * tt
                u1 = u0 + 1
                g_start(sg_v, u1, rows_b, semg_b)
                g_wait(sg_v, u0, rows_a, semg_a)
                scale(u0, rows_a)
                s_start(dg_v, u0, rows_a, sems_a)
                g_wait(sg_v, u1, rows_b, semg_b)
                scale(u1, rows_b)
                s_start(dg_v, u1, rows_b, sems_b)
                s_wait(dg_v, u0, rows_a, sems_a)
                g_start(sg_v, u0 + 2, rows_a, semg_a)
                s_wait(dg_v, u1, rows_b, sems_b)
                return carry
            lax.fori_loop(0, (GC - 1) // 2, pair, 0)

            ul = GC - 1
            g_wait(sg_v, ul, rows_a, semg_a)
            scale(ul, rows_a)
            s_start(dg_v, ul, rows_a, sems_a)
            s_wait(dg_v, ul, rows_a, sems_a)
            if g + 1 < NG:
                idx_wait(g + 1, 1 - p, semi)
        plsc.subcore_barrier()

        pltpu.sync_copy(s_sh.at[pl.ds(base, S_ROWS)],
                        s_out.at[c, pl.ds(base, S_ROWS)])

    return k(ei4, w4, hp)


def _mm_h(x, W_conv, deg0, deg1):
    def body(x_ref, w_ref, d0_ref, d1_ref, hp_ref, dis_ref):
        deg = d0_ref[...] + d1_ref[...] + 1.0
        dis = lax.rsqrt(deg)
        hp_ref[...] = jnp.dot(x_ref[...], w_ref[...],
                              preferred_element_type=jnp.float32) * dis
        dis_ref[...] = dis

    return pl.pallas_call(
        body,
        grid=(N // 2000,),
        in_specs=[
            pl.BlockSpec((2000, D), lambda i: (i, 0)),
            pl.BlockSpec((D, D), lambda i: (0, 0)),
            pl.BlockSpec((2000, 1), lambda i: (i, 0)),
            pl.BlockSpec((2000, 1), lambda i: (i, 0)),
        ],
        out_specs=[
            pl.BlockSpec((2000, D), lambda i: (i, 0)),
            pl.BlockSpec((2000, 1), lambda i: (i, 0)),
        ],
        out_shape=[
            jax.ShapeDtypeStruct((N, D), jnp.float32),
            jax.ShapeDtypeStruct((N, 1), jnp.float32),
        ],
    )(x, W_conv, deg0, deg1)


def _epilogue(s_part, hp, dis, b_conv, W_out, b_out):
    def body(s0_ref, s1_ref, hp_ref, dis_ref, bc_ref, wo_ref, bo_ref, o_ref):
        agg = dis_ref[...] * (s0_ref[0] + s1_ref[0] + hp_ref[...]) \
            + bc_ref[...]
        emb = jnp.where(agg >= 0, agg, 0.01 * agg)
        o_ref[...] = jnp.dot(emb, wo_ref[...],
                             preferred_element_type=jnp.float32) + bo_ref[...]

    return pl.pallas_call(
        body,
        grid=(N // 2000,),
        in_specs=[
            pl.BlockSpec((1, 2000, D), lambda i: (0, i, 0)),
            pl.BlockSpec((1, 2000, D), lambda i: (1, i, 0)),
            pl.BlockSpec((2000, D), lambda i: (i, 0)),
            pl.BlockSpec((2000, 1), lambda i: (i, 0)),
            pl.BlockSpec((1, D), lambda i: (0, 0)),
            pl.BlockSpec((D, D_OUT), lambda i: (0, 0)),
            pl.BlockSpec((1, D_OUT), lambda i: (0, 0)),
        ],
        out_specs=pl.BlockSpec((2000, D_OUT), lambda i: (i, 0)),
        out_shape=jax.ShapeDtypeStruct((N, D_OUT), jnp.float32),
    )(s_part, s_part, hp, dis, b_conv, W_out, b_out)


def kernel(x, edge_index, edge_weight, W_conv, b_conv, W_out, b_out):
    ei4 = edge_index.astype(jnp.int32).reshape(2, NW, NG, GC, K)
    w4 = edge_weight.astype(jnp.float32).reshape(NW, NG, GC, K)

    deg_p = _deg_kernel(ei4, w4)
    deg0 = deg_p[0].reshape(NPD, 1)
    deg1 = deg_p[1].reshape(NPD, 1)
    hp, dis = _mm_h(x, W_conv, deg0, deg1)
    s_part = _agg_kernel(ei4, w4, hp)
    out = _epilogue(
        s_part, hp, dis,
        b_conv.reshape(1, D), W_out, b_out.reshape(1, D_OUT))
    return out


# speedup vs baseline: 1.2528x; 1.0043x over previous
import functools

import jax
import jax.numpy as jnp
from jax import lax
from jax.experimental import pallas as pl
from jax.experimental.pallas import tpu as pltpu
from jax.experimental.pallas import tpu_sc as plsc

N = 10000
E = 320000
D = 128
D_OUT = 64
NC = 2
NS = 16
NW = NC * NS
EPW = E // NW
K = 80
C = EPW // K
GC = 25
NG = C // GC
NPD = 10240
DEG_ROWS = NPD // NS
S_ROWS = NPD // NS

_MESH = dict(core_axis_name="c", subcore_axis_name="s",
             num_cores=NC, num_subcores=NS)
_PARAMS = None


def _sc_params():
    return pltpu.CompilerParams(needs_layout_passes=False)


def _deg_kernel(ei4, w4):
    @functools.partial(
        pl.kernel,
        out_type=jax.ShapeDtypeStruct((NC, NPD), jnp.float32),
        mesh=plsc.VectorSubcoreMesh(**_MESH),
        compiler_params=_sc_params(),
        scratch_types=[
            pltpu.VMEM((NG, GC, K), jnp.int32),
            pltpu.VMEM((NG, GC, K), jnp.float32),
            pltpu.VMEM((DEG_ROWS,), jnp.float32),
            pltpu.VMEM_SHARED((NPD,), jnp.float32),
            pltpu.SemaphoreType.DMA,
        ],
    )
    def k(ei4_h, w4_h, deg_out, dst_v, w_v, nbuf, deg_sh, sem):
        c = lax.axis_index("c")
        s = lax.axis_index("s")
        wid = s * NC + c
        base = s * DEG_ROWS

        z16 = jnp.zeros((16,), jnp.float32)

        def zero_nbuf(i, carry):
            nbuf[pl.ds(i * 16, 16)] = z16
            return carry
        lax.fori_loop(0, DEG_ROWS // 16, zero_nbuf, 0)
        pltpu.sync_copy(nbuf, deg_sh.at[pl.ds(base, DEG_ROWS)])

        pltpu.sync_copy(ei4_h.at[1, wid], dst_v)
        pltpu.sync_copy(w4_h.at[wid], w_v)
        plsc.subcore_barrier()

        def deg_group(i, carry):
            g = i // 5
            t = i % 5
            for u in range(5):
                j = t * 5 + u
                pltpu.async_copy(
                    w_v.at[g, j], deg_sh.at[dst_v.at[g, j]], sem, add=True)
            for u in range(5):
                j = t * 5 + u
                pltpu.make_async_copy(
                    w_v.at[g, j], deg_sh.at[dst_v.at[g, j]], sem).wait()
            return carry
        lax.fori_loop(0, C // 5, deg_group, 0)
        plsc.subcore_barrier()

        pltpu.sync_copy(deg_sh.at[pl.ds(base, DEG_ROWS)],
                        deg_out.at[c, pl.ds(base, DEG_ROWS)])

    return k(ei4, w4)


def _agg_kernel(ei4, w4, hp):
    @functools.partial(
        pl.kernel,
        out_type=jax.ShapeDtypeStruct((NC, NPD, D), jnp.float32),
        mesh=plsc.VectorSubcoreMesh(**_MESH),
        compiler_params=_sc_params(),
        scratch_types=[
            pltpu.VMEM((2, GC, K), jnp.int32),
            pltpu.VMEM((2, GC, K), jnp.int32),
            pltpu.VMEM((2, GC, K), jnp.float32),
            pltpu.VMEM((K, D), jnp.float32),
            pltpu.VMEM((K, D), jnp.float32),
            pltpu.VMEM_SHARED((NPD, D), jnp.float32),
            pltpu.SemaphoreType.DMA,
            pltpu.SemaphoreType.DMA,
            pltpu.SemaphoreType.DMA,
            pltpu.SemaphoreType.DMA,
            pltpu.SemaphoreType.DMA,
        ],
    )
    def k(ei4_h, w4_h, hp_hbm, s_out,
          sg2, dg2, wg2, rows_a, rows_b, s_sh,
          semg_a, semg_b, sems_a, sems_b, semi):
        c = lax.axis_index("c")
        s = lax.axis_index("s")
        wid = s * NC + c
        base = s * S_ROWS

        z16 = jnp.zeros((16,), jnp.float32)

        def g_start(sg_v, u, buf, sem):
            pltpu.async_copy(hp_hbm.at[sg_v.at[u]], buf, sem)

        def g_wait(sg_v, u, buf, sem):
            pltpu.make_async_copy(hp_hbm.at[sg_v.at[u]], buf, sem).wait()

        def s_start(dg_v, u, buf, sem):
            pltpu.async_copy(buf, s_sh.at[dg_v.at[u]], sem, add=True)

        def s_wait(dg_v, u, buf, sem):
            pltpu.make_async_copy(buf, s_sh.at[dg_v.at[u]], sem).wait()

        def make_scale(wg_v):
            def scale(u, buf):
                def row(i, carry):
                    ii = jnp.full((16,), i, jnp.int32)
                    uu = jnp.full((16,), u, jnp.int32)
                    cb = plsc.load_gather(wg_v, [uu, ii])
                    for t in range(D // 16):
                        buf[i, pl.ds(t * 16, 16)] = \
                            buf[i, pl.ds(t * 16, 16)] * cb
                    return carry
                lax.fori_loop(0, K, row, 0)
            return scale

        def idx_prefetch(g, p, sem):
            pltpu.async_copy(ei4_h.at[0, wid, g], sg2.at[p], sem)
            pltpu.async_copy(ei4_h.at[1, wid, g], dg2.at[p], sem)
            pltpu.async_copy(w4_h.at[wid, g], wg2.at[p], sem)

        def idx_wait(g, p, sem):
            pltpu.make_async_copy(ei4_h.at[0, wid, g], sg2.at[p], sem).wait()
            pltpu.make_async_copy(ei4_h.at[1, wid, g], dg2.at[p], sem).wait()
            pltpu.make_async_copy(w4_h.at[wid, g], wg2.at[p], sem).wait()

        idx_prefetch(0, 0, semi)

        def zero_rows(i, carry):
            for t in range(D // 16):
                rows_b[i, pl.ds(t * 16, 16)] = z16
            return carry
        lax.fori_loop(0, K, zero_rows, 0)
        idx_wait(0, 0, semi)
        g_start(sg2.at[0], 0, rows_a, semg_a)
        for kk in range(S_ROWS // K):
            pltpu.sync_copy(rows_b, s_sh.at[pl.ds(base + kk * K, K)])
        plsc.subcore_barrier()

        for g in range(NG):
            p = g % 2
            sg_v, dg_v, wg_v = sg2.at[p], dg2.at[p], wg2.at[p]
            scale = make_scale(wg_v)
            if g + 1 < NG:
                idx_prefetch(g + 1, 1 - p, semi)
            if g > 0:
                g_start(sg_v, 0, rows_a, semg_a)

            def pair(tt, carry):
                u0 = 2 * tt
                u1 = u0 + 1
                g_start(sg_v, u1, rows_b, semg_b)
                g_wait(sg_v, u0, rows_a, semg_a)
                scale(u0, rows_a)
                s_start(dg_v, u0, rows_a, sems_a)
                g_wait(sg_v, u1, rows_b, semg_b)
                scale(u1, rows_b)
                s_start(dg_v, u1, rows_b, sems_b)
                s_wait(dg_v, u0, rows_a, sems_a)
                g_start(sg_v, u0 + 2, rows_a, semg_a)
                s_wait(dg_v, u1, rows_b, sems_b)
                return carry
            lax.fori_loop(0, (GC - 1) // 2, pair, 0)

            ul = GC - 1
            g_wait(sg_v, ul, rows_a, semg_a)
            scale(ul, rows_a)
            s_start(dg_v, ul, rows_a, sems_a)
            s_wait(dg_v, ul, rows_a, sems_a)
            if g + 1 < NG:
                idx_wait(g + 1, 1 - p, semi)
        plsc.subcore_barrier()

        pltpu.sync_copy(s_sh.at[pl.ds(base, S_ROWS)],
                        s_out.at[c, pl.ds(base, S_ROWS)])

    return k(ei4, w4, hp)


def _mm_h(x, W_conv, deg0, deg1):
    def body(x_ref, w_ref, d0_ref, d1_ref, hp_ref, dis_ref):
        deg = d0_ref[...] + d1_ref[...] + 1.0
        dis = lax.rsqrt(deg)
        hp_ref[...] = jnp.dot(x_ref[...], w_ref[...],
                              preferred_element_type=jnp.float32) * dis
        dis_ref[...] = dis

    return pl.pallas_call(
        body,
        grid=(N // 2000,),
        in_specs=[
            pl.BlockSpec((2000, D), lambda i: (i, 0)),
            pl.BlockSpec((D, D), lambda i: (0, 0)),
            pl.BlockSpec((2000, 1), lambda i: (i, 0)),
            pl.BlockSpec((2000, 1), lambda i: (i, 0)),
        ],
        out_specs=[
            pl.BlockSpec((2000, D), lambda i: (i, 0)),
            pl.BlockSpec((2000, 1), lambda i: (i, 0)),
        ],
        out_shape=[
            jax.ShapeDtypeStruct((N, D), jnp.float32),
            jax.ShapeDtypeStruct((N, 1), jnp.float32),
        ],
    )(x, W_conv, deg0, deg1)


def _epilogue(s_part, hp, dis, b_conv, W_out, b_out):
    def body(s0_ref, s1_ref, hp_ref, dis_ref, bc_ref, wo_ref, bo_ref, o_ref):
        agg = dis_ref[...] * (s0_ref[0] + s1_ref[0] + hp_ref[...]) \
            + bc_ref[...]
        emb = jnp.where(agg >= 0, agg, 0.01 * agg)
        o_ref[...] = jnp.dot(emb, wo_ref[...],
                             preferred_element_type=jnp.float32) + bo_ref[...]

    return pl.pallas_call(
        body,
        grid=(N // 2000,),
        in_specs=[
            pl.BlockSpec((1, 2000, D), lambda i: (0, i, 0)),
            pl.BlockSpec((1, 2000, D), lambda i: (1, i, 0)),
            pl.BlockSpec((2000, D), lambda i: (i, 0)),
            pl.BlockSpec((2000, 1), lambda i: (i, 0)),
            pl.BlockSpec((1, D), lambda i: (0, 0)),
            pl.BlockSpec((D, D_OUT), lambda i: (0, 0)),
            pl.BlockSpec((1, D_OUT), lambda i: (0, 0)),
        ],
        out_specs=pl.BlockSpec((2000, D_OUT), lambda i: (i, 0)),
        out_shape=jax.ShapeDtypeStruct((N, D_OUT), jnp.float32),
    )(s_part, s_part, hp, dis, b_conv, W_out, b_out)


def kernel(x, edge_index, edge_weight, W_conv, b_conv, W_out, b_out):
    ei4 = edge_index.astype(jnp.int32).reshape(2, NW, NG, GC, K)
    w4 = edge_weight.astype(jnp.float32).reshape(NW, NG, GC, K)

    deg_p = _deg_kernel(ei4, w4)
    deg0 = deg_p[0].reshape(NPD, 1)
    deg1 = deg_p[1].reshape(NPD, 1)
    hp, dis = _mm_h(x, W_conv, deg0, deg1)
    s_part = _agg_kernel(ei4, w4, hp)
    out = _epilogue(
        s_part, hp, dis,
        b_conv.reshape(1, D), W_out, b_out.reshape(1, D_OUT))
    return out


# speedup vs baseline: 1.2536x; 1.0006x over previous
import functools

import jax
import jax.numpy as jnp
from jax import lax
from jax.experimental import pallas as pl
from jax.experimental.pallas import tpu as pltpu
from jax.experimental.pallas import tpu_sc as plsc

N = 10000
E = 320000
D = 128
D_OUT = 64
NC = 2
NS = 16
NW = NC * NS
EPW = E // NW
K = 80
C = EPW // K
GC = 25
NG = C // GC
NPD = 10240
DEG_ROWS = NPD // NS
S_ROWS = NPD // NS

_MESH = dict(core_axis_name="c", subcore_axis_name="s",
             num_cores=NC, num_subcores=NS)


def _sc_params():
    return pltpu.CompilerParams(needs_layout_passes=False)


def _deg_kernel(ei4, w4):
    @functools.partial(
        pl.kernel,
        out_type=jax.ShapeDtypeStruct((NC, NPD), jnp.float32),
        mesh=plsc.VectorSubcoreMesh(**_MESH),
        compiler_params=_sc_params(),
        scratch_types=[
            pltpu.VMEM((NG, GC, K), jnp.int32),
            pltpu.VMEM((NG, GC, K), jnp.float32),
            pltpu.VMEM((DEG_ROWS,), jnp.float32),
            pltpu.VMEM_SHARED((NPD,), jnp.float32),
            pltpu.SemaphoreType.DMA,
        ],
    )
    def k(ei4_h, w4_h, deg_out, dst_v, w_v, nbuf, deg_sh, sem):
        c = lax.axis_index("c")
        s = lax.axis_index("s")
        wid = s * NC + c
        base = s * DEG_ROWS

        z16 = jnp.zeros((16,), jnp.float32)

        def zero_nbuf(i, carry):
            nbuf[pl.ds(i * 16, 16)] = z16
            return carry
        lax.fori_loop(0, DEG_ROWS // 16, zero_nbuf, 0)
        pltpu.sync_copy(nbuf, deg_sh.at[pl.ds(base, DEG_ROWS)])

        pltpu.sync_copy(ei4_h.at[1, wid], dst_v)
        pltpu.sync_copy(w4_h.at[wid], w_v)
        plsc.subcore_barrier()

        def deg_group(i, carry):
            g = i // 5
            t = i % 5
            for u in range(5):
                j = t * 5 + u
                pltpu.async_copy(
                    w_v.at[g, j], deg_sh.at[dst_v.at[g, j]], sem, add=True)
            for u in range(5):
                j = t * 5 + u
                pltpu.make_async_copy(
                    w_v.at[g, j], deg_sh.at[dst_v.at[g, j]], sem).wait()
            return carry
        lax.fori_loop(0, C // 5, deg_group, 0)
        plsc.subcore_barrier()

        pltpu.sync_copy(deg_sh.at[pl.ds(base, DEG_ROWS)],
                        deg_out.at[c, pl.ds(base, DEG_ROWS)])

    return k(ei4, w4)


def _agg_kernel(ei4, w4, hp):
    @functools.partial(
        pl.kernel,
        out_type=jax.ShapeDtypeStruct((NC, NPD, D), jnp.float32),
        mesh=plsc.VectorSubcoreMesh(**_MESH),
        compiler_params=_sc_params(),
        scratch_types=[
            pltpu.VMEM((2, GC, K), jnp.int32),
            pltpu.VMEM((2, GC, K), jnp.int32),
            pltpu.VMEM((2, GC, K), jnp.float32),
            pltpu.VMEM((K, D), jnp.float32),
            pltpu.VMEM((K, D), jnp.float32),
            pltpu.VMEM_SHARED((NPD, D), jnp.float32),
            pltpu.SemaphoreType.DMA,
            pltpu.SemaphoreType.DMA,
            pltpu.SemaphoreType.DMA,
            pltpu.SemaphoreType.DMA,
            pltpu.SemaphoreType.DMA,
        ],
    )
    def k(ei4_h, w4_h, hp_hbm, s_out,
          sg2, dg2, wg2, rows_a, rows_b, s_sh,
          semg_a, semg_b, sems_a, sems_b, semi):
        c = lax.axis_index("c")
        s = lax.axis_index("s")
        wid = s * NC + c
        base = s * S_ROWS

        z16 = jnp.zeros((16,), jnp.float32)

        def g_start(sg_v, u, buf, sem):
            pltpu.async_copy(hp_hbm.at[sg_v.at[u]], buf, sem)

        def g_wait(sg_v, u, buf, sem):
            pltpu.make_async_copy(hp_hbm.at[sg_v.at[u]], buf, sem).wait()

        def s_start(dg_v, u, buf, sem):
            pltpu.async_copy(buf, s_sh.at[dg_v.at[u]], sem, add=True)

        def s_wait(dg_v, u, buf, sem):
            pltpu.make_async_copy(buf, s_sh.at[dg_v.at[u]], sem).wait()

        def make_scale(wg_v):
            def scale(u, buf):
                def row(i, carry):
                    ii = jnp.full((16,), i, jnp.int32)
                    uu = jnp.full((16,), u, jnp.int32)
                    cb = plsc.load_gather(wg_v, [uu, ii])
                    for t in range(D // 16):
                        buf[i, pl.ds(t * 16, 16)] = \
                            buf[i, pl.ds(t * 16, 16)] * cb
                    return carry
                lax.fori_loop(0, K, row, 0)
            return scale

        def idx_prefetch(g, p, sem):
            pltpu.async_copy(ei4_h.at[0, wid, g], sg2.at[p], sem)
            pltpu.async_copy(ei4_h.at[1, wid, g], dg2.at[p], sem)
            pltpu.async_copy(w4_h.at[wid, g], wg2.at[p], sem)

        def idx_wait(g, p, sem):
            pltpu.make_async_copy(ei4_h.at[0, wid, g], sg2.at[p], sem).wait()
            pltpu.make_async_copy(ei4_h.at[1, wid, g], dg2.at[p], sem).wait()
            pltpu.make_async_copy(w4_h.at[wid, g], wg2.at[p], sem).wait()

        idx_prefetch(0, 0, semi)

        def zero_rows(i, carry):
            for t in range(D // 16):
                rows_b[i, pl.ds(t * 16, 16)] = z16
            return carry
        lax.fori_loop(0, K, zero_rows, 0)
        idx_wait(0, 0, semi)
        g_start(sg2.at[0], 0, rows_a, semg_a)
        for kk in range(S_ROWS // K):
            pltpu.sync_copy(rows_b, s_sh.at[pl.ds(base + kk * K, K)])
        plsc.subcore_barrier()

        for g in range(NG):
            p = g % 2
            sg_v, dg_v, wg_v = sg2.at[p], dg2.at[p], wg2.at[p]
            scale = make_scale(wg_v)
            if g + 1 < NG:
                idx_prefetch(g + 1, 1 - p, semi)
            if g > 0:
                g_start(sg_v, 0, rows_a, semg_a)

            def pair(tt, carry):
                u0 = 2 * tt
                u1 = u0 + 1
                g_start(sg_v, u1, rows_b, semg_b)
                g_wait(sg_v, u0, rows_a, semg_a)
                scale(u0, rows_a)
                s_start(dg_v, u0, rows_a, sems_a)
                g_wait(sg_v, u1, rows_b, semg_b)
                scale(u1, rows_b)
                s_start(dg_v, u1, rows_b, sems_b)
                s_wait(dg_v, u0, rows_a, sems_a)
                g_start(sg_v, u0 + 2, rows_a, semg_a)
                s_wait(dg_v, u1, rows_b, sems_b)
                return carry
            lax.fori_loop(0, (GC - 1) // 2, pair, 0)

            ul = GC - 1
            g_wait(sg_v, ul, rows_a, semg_a)
            scale(ul, rows_a)
            s_start(dg_v, ul, rows_a, sems_a)
            s_wait(dg_v, ul, rows_a, sems_a)
            if g + 1 < NG:
                idx_wait(g + 1, 1 - p, semi)
        plsc.subcore_barrier()

        pltpu.sync_copy(s_sh.at[pl.ds(base, S_ROWS)],
                        s_out.at[c, pl.ds(base, S_ROWS)])

    return k(ei4, w4, hp)


def _mm_h(x, W_conv, deg0, deg1):
    def body(x_ref, w_ref, d0_ref, d1_ref, hp_ref, dis_ref):
        deg = d0_ref[...] + d1_ref[...] + 1.0
        dis = lax.rsqrt(deg)
        hp_ref[...] = jnp.dot(x_ref[...], w_ref[...],
                              preferred_element_type=jnp.float32) * dis
        dis_ref[...] = dis

    return pl.pallas_call(
        body,
        grid=(N // 2000,),
        in_specs=[
            pl.BlockSpec((2000, D), lambda i: (i, 0)),
            pl.BlockSpec((D, D), lambda i: (0, 0)),
            pl.BlockSpec((2000, 1), lambda i: (i, 0)),
            pl.BlockSpec((2000, 1), lambda i: (i, 0)),
        ],
        out_specs=[
            pl.BlockSpec((2000, D), lambda i: (i, 0)),
            pl.BlockSpec((2000, 1), lambda i: (i, 0)),
        ],
        out_shape=[
            jax.ShapeDtypeStruct((N, D), jnp.float32),
            jax.ShapeDtypeStruct((N, 1), jnp.float32),
        ],
    )(x, W_conv, deg0, deg1)


def _epilogue(s_part, hp, dis, b_conv, W_out, b_out):
    def body(s0_ref, s1_ref, hp_ref, dis_ref, bc_ref, wo_ref, bo_ref, o_ref):
        agg = dis_ref[...] * (s0_ref[0] + s1_ref[0] + hp_ref[...]) \
            + bc_ref[...]
        emb = jnp.where(agg >= 0, agg, 0.01 * agg)
        o_ref[...] = jnp.dot(emb, wo_ref[...],
                             preferred_element_type=jnp.float32) + bo_ref[...]

    return pl.pallas_call(
        body,
        grid=(N // 2000,),
        in_specs=[
            pl.BlockSpec((1, 2000, D), lambda i: (0, i, 0)),
            pl.BlockSpec((1, 2000, D), lambda i: (1, i, 0)),
            pl.BlockSpec((2000, D), lambda i: (i, 0)),
            pl.BlockSpec((2000, 1), lambda i: (i, 0)),
            pl.BlockSpec((1, D), lambda i: (0, 0)),
            pl.BlockSpec((D, D_OUT), lambda i: (0, 0)),
            pl.BlockSpec((1, D_OUT), lambda i: (0, 0)),
        ],
        out_specs=pl.BlockSpec((2000, D_OUT), lambda i: (i, 0)),
        out_shape=jax.ShapeDtypeStruct((N, D_OUT), jnp.float32),
    )(s_part, s_part, hp, dis, b_conv, W_out, b_out)


def kernel(x, edge_index, edge_weight, W_conv, b_conv, W_out, b_out):
    ei4 = edge_index.astype(jnp.int32).reshape(2, NW, NG, GC, K)
    w4 = edge_weight.astype(jnp.float32).reshape(NW, NG, GC, K)

    deg_p = _deg_kernel(ei4, w4)
    deg0 = deg_p[0].reshape(NPD, 1)
    deg1 = deg_p[1].reshape(NPD, 1)
    hp, dis = _mm_h(x, W_conv, deg0, deg1)
    s_part = _agg_kernel(ei4, w4, hp)
    out = _epilogue(
        s_part, hp, dis,
        b_conv.reshape(1, D), W_out, b_out.reshape(1, D_OUT))
    return out


# speedup vs baseline: 1.2726x; 1.0152x over previous
import functools

import jax
import jax.numpy as jnp
from jax import lax
from jax.experimental import pallas as pl
from jax.experimental.pallas import tpu as pltpu
from jax.experimental.pallas import tpu_sc as plsc

N = 10000
E = 320000
D = 128
D_OUT = 64
NC = 2
NS = 16
NW = NC * NS
EPW = E // NW
K = 80
C = EPW // K
GC = 25
NG = C // GC
NPD = 10240
DEG_ROWS = NPD // NS
S_ROWS = NPD // NS

_MESH = dict(core_axis_name="c", subcore_axis_name="s",
             num_cores=NC, num_subcores=NS)


def _sc_params():
    return pltpu.CompilerParams(needs_layout_passes=False)


def _deg_kernel(ei4, w4):
    @functools.partial(
        pl.kernel,
        out_type=jax.ShapeDtypeStruct((NC, NPD), jnp.float32),
        mesh=plsc.VectorSubcoreMesh(**_MESH),
        compiler_params=_sc_params(),
        scratch_types=[
            pltpu.VMEM((NG, GC, K), jnp.int32),
            pltpu.VMEM((NG, GC, K), jnp.float32),
            pltpu.VMEM((DEG_ROWS,), jnp.float32),
            pltpu.VMEM_SHARED((NPD,), jnp.float32),
            pltpu.SemaphoreType.DMA,
        ],
    )
    def k(ei4_h, w4_h, deg_out, dst_v, w_v, nbuf, deg_sh, sem):
        c = lax.axis_index("c")
        s = lax.axis_index("s")
        wid = s * NC + c
        base = s * DEG_ROWS

        z16 = jnp.zeros((16,), jnp.float32)

        def zero_nbuf(i, carry):
            nbuf[pl.ds(i * 16, 16)] = z16
            return carry
        lax.fori_loop(0, DEG_ROWS // 16, zero_nbuf, 0)
        pltpu.sync_copy(nbuf, deg_sh.at[pl.ds(base, DEG_ROWS)])

        pltpu.sync_copy(ei4_h.at[1, wid], dst_v)
        pltpu.sync_copy(w4_h.at[wid], w_v)
        plsc.subcore_barrier()

        def deg_group(i, carry):
            g = i // 5
            t = i % 5
            for u in range(5):
                j = t * 5 + u
                pltpu.async_copy(
                    w_v.at[g, j], deg_sh.at[dst_v.at[g, j]], sem, add=True)
            for u in range(5):
                j = t * 5 + u
                pltpu.make_async_copy(
                    w_v.at[g, j], deg_sh.at[dst_v.at[g, j]], sem).wait()
            return carry
        lax.fori_loop(0, C // 5, deg_group, 0)
        plsc.subcore_barrier()

        pltpu.sync_copy(deg_sh.at[pl.ds(base, DEG_ROWS)],
                        deg_out.at[c, pl.ds(base, DEG_ROWS)])

    return k(ei4, w4)


def _agg_kernel(ei4, w4, hp):
    @functools.partial(
        pl.kernel,
        out_type=jax.ShapeDtypeStruct((NC, NPD, D), jnp.float32),
        mesh=plsc.VectorSubcoreMesh(**_MESH),
        compiler_params=_sc_params(),
        scratch_types=[
            pltpu.VMEM((2, GC, K), jnp.int32),
            pltpu.VMEM((2, GC, K), jnp.int32),
            pltpu.VMEM((2, GC, K), jnp.float32),
            pltpu.VMEM((K, D), jnp.float32),
            pltpu.VMEM((K, D), jnp.float32),
            pltpu.VMEM_SHARED((NPD, D), jnp.float32),
            pltpu.SemaphoreType.DMA,
            pltpu.SemaphoreType.DMA,
            pltpu.SemaphoreType.DMA,
            pltpu.SemaphoreType.DMA,
            pltpu.SemaphoreType.DMA,
        ],
    )
    def k(ei4_h, w4_h, hp_hbm, s_out,
          sg2, dg2, wg2, rows_a, rows_b, s_sh,
          semg_a, semg_b, sems_a, sems_b, semi):
        c = lax.axis_index("c")
        s = lax.axis_index("s")
        wid = s * NC + c
        base = s * S_ROWS

        z16 = jnp.zeros((16,), jnp.float32)

        def g_start(sg_v, u, buf, sem):
            pltpu.async_copy(hp_hbm.at[sg_v.at[u]], buf, sem)

        def g_wait(sg_v, u, buf, sem):
            pltpu.make_async_copy(hp_hbm.at[sg_v.at[u]], buf, sem).wait()

        def s_start(dg_v, u, buf, sem):
            pltpu.async_copy(buf, s_sh.at[dg_v.at[u]], sem, add=True)

        def s_wait(dg_v, u, buf, sem):
            pltpu.make_async_copy(buf, s_sh.at[dg_v.at[u]], sem).wait()

        def make_scale(wg_v):
            def scale(u, buf):
                def row(i, carry):
                    ii = jnp.full((16,), i, jnp.int32)
                    uu = jnp.full((16,), u, jnp.int32)
                    cb = plsc.load_gather(wg_v, [uu, ii])
                    for t in range(D // 16):
                        buf[i, pl.ds(t * 16, 16)] = \
                            buf[i, pl.ds(t * 16, 16)] * cb
                    return carry
                lax.fori_loop(0, K, row, 0)
            return scale

        def idx_prefetch(g, p, sem):
            pltpu.async_copy(ei4_h.at[0, wid, g], sg2.at[p], sem)
            pltpu.async_copy(ei4_h.at[1, wid, g], dg2.at[p], sem)
            pltpu.async_copy(w4_h.at[wid, g], wg2.at[p], sem)

        def idx_wait(g, p, sem):
            pltpu.make_async_copy(ei4_h.at[0, wid, g], sg2.at[p], sem).wait()
            pltpu.make_async_copy(ei4_h.at[1, wid, g], dg2.at[p], sem).wait()
            pltpu.make_async_copy(w4_h.at[wid, g], wg2.at[p], sem).wait()

        idx_prefetch(0, 0, semi)

        def zero_rows(i, carry):
            for t in range(D // 16):
                rows_b[i, pl.ds(t * 16, 16)] = z16
            return carry
        lax.fori_loop(0, K, zero_rows, 0)
        idx_wait(0, 0, semi)
        g_start(sg2.at[0], 0, rows_a, semg_a)
        for kk in range(S_ROWS // K):
            pltpu.sync_copy(rows_b, s_sh.at[pl.ds(base + kk * K, K)])
        plsc.subcore_barrier()

        for g in range(NG):
            p = g % 2
            sg_v, dg_v, wg_v = sg2.at[p], dg2.at[p], wg2.at[p]
            scale = make_scale(wg_v)
            if p == 0:
                rx, ry = rows_a, rows_b
                gx, gy = semg_a, semg_b
                sx, sy = sems_a, sems_b
            else:
                rx, ry = rows_b, rows_a
                gx, gy = semg_b, semg_a
                sx, sy = sems_b, sems_a
            if g + 1 < NG:
                idx_prefetch(g + 1, 1 - p, semi)

            def pair(tt, carry):
                u0 = 2 * tt
                u1 = u0 + 1
                g_start(sg_v, u1, ry, gy)
                g_wait(sg_v, u0, rx, gx)
                scale(u0, rx)
                s_start(dg_v, u0, rx, sx)
                g_wait(sg_v, u1, ry, gy)
                scale(u1, ry)
                s_start(dg_v, u1, ry, sy)
                s_wait(dg_v, u0, rx, sx)
                g_start(sg_v, u0 + 2, rx, gx)
                s_wait(dg_v, u1, ry, sy)
                return carry
            lax.fori_loop(0, (GC - 1) // 2, pair, 0)

            ul = GC - 1
            g_wait(sg_v, ul, rx, gx)
            scale(ul, rx)
            s_start(dg_v, ul, rx, sx)
            if g + 1 < NG:
                idx_wait(g + 1, 1 - p, semi)
                g_start(sg2.at[1 - p], 0, ry, gy)
            s_wait(dg_v, ul, rx, sx)
        plsc.subcore_barrier()

        pltpu.sync_copy(s_sh.at[pl.ds(base, S_ROWS)],
                        s_out.at[c, pl.ds(base, S_ROWS)])

    return k(ei4, w4, hp)


def _mm_h(x, W_conv, deg0, deg1):
    def body(x_ref, w_ref, d0_ref, d1_ref, hp_ref, dis_ref):
        deg = d0_ref[...] + d1_ref[...] + 1.0
        dis = lax.rsqrt(deg)
        hp_ref[...] = jnp.dot(x_ref[...], w_ref[...],
                              preferred_element_type=jnp.float32) * dis
        dis_ref[...] = dis

    return pl.pallas_call(
        body,
        grid=(N // 2000,),
        in_specs=[
            pl.BlockSpec((2000, D), lambda i: (i, 0)),
            pl.BlockSpec((D, D), lambda i: (0, 0)),
            pl.BlockSpec((2000, 1), lambda i: (i, 0)),
            pl.BlockSpec((2000, 1), lambda i: (i, 0)),
        ],
        out_specs=[
            pl.BlockSpec((2000, D), lambda i: (i, 0)),
            pl.BlockSpec((2000, 1), lambda i: (i, 0)),
        ],
        out_shape=[
            jax.ShapeDtypeStruct((N, D), jnp.float32),
            jax.ShapeDtypeStruct((N, 1), jnp.float32),
        ],
    )(x, W_conv, deg0, deg1)


def _epilogue(s_part, hp, dis, b_conv, W_out, b_out):
    def body(s0_ref, s1_ref, hp_ref, dis_ref, bc_ref, wo_ref, bo_ref, o_ref):
        agg = dis_ref[...] * (s0_ref[0] + s1_ref[0] + hp_ref[...]) \
            + bc_ref[...]
        emb = jnp.where(agg >= 0, agg, 0.01 * agg)
        o_ref[...] = jnp.dot(emb, wo_ref[...],
                             preferred_element_type=jnp.float32) + bo_ref[...]

    return pl.pallas_call(
        body,
        grid=(N // 2000,),
        in_specs=[
            pl.BlockSpec((1, 2000, D), lambda i: (0, i, 0)),
            pl.BlockSpec((1, 2000, D), lambda i: (1, i, 0)),
            pl.BlockSpec((2000, D), lambda i: (i, 0)),
            pl.BlockSpec((2000, 1), lambda i: (i, 0)),
            pl.BlockSpec((1, D), lambda i: (0, 0)),
            pl.BlockSpec((D, D_OUT), lambda i: (0, 0)),
            pl.BlockSpec((1, D_OUT), lambda i: (0, 0)),
        ],
        out_specs=pl.BlockSpec((2000, D_OUT), lambda i: (i, 0)),
        out_shape=jax.ShapeDtypeStruct((N, D_OUT), jnp.float32),
    )(s_part, s_part, hp, dis, b_conv, W_out, b_out)


def kernel(x, edge_index, edge_weight, W_conv, b_conv, W_out, b_out):
    ei4 = edge_index.astype(jnp.int32).reshape(2, NW, NG, GC, K)
    w4 = edge_weight.astype(jnp.float32).reshape(NW, NG, GC, K)

    deg_p = _deg_kernel(ei4, w4)
    deg0 = deg_p[0].reshape(NPD, 1)
    deg1 = deg_p[1].reshape(NPD, 1)
    hp, dis = _mm_h(x, W_conv, deg0, deg1)
    s_part = _agg_kernel(ei4, w4, hp)
    out = _epilogue(
        s_part, hp, dis,
        b_conv.reshape(1, D), W_out, b_out.reshape(1, D_OUT))
    return out
